# Initial kernel scaffold; baseline (speedup 1.0000x reference)
import jax
import jax.numpy as jnp
from jax.experimental import pallas as pl


def kernel(x, adj, W1, b1, W2, b2):
    raise NotImplementedError("write your pallas kernel here")


# speedup vs baseline: 3.4710x; 3.4710x over previous
import functools

import jax
import jax.numpy as jnp
from jax import lax
from jax.experimental import pallas as pl
from jax.experimental.pallas import tpu as pltpu
from jax.experimental.pallas import tpu_sc as plsc

N = 10000
E = 160000
NFEAT = 256
NHID = 256
NCLASS = 64

NC = 2
NS = 16
CH = 128

NP = 10112
RPT = NP // NS

EPT1 = 10112
EPAD1 = NS * EPT1

EPT2 = 5120
EPAD2 = NC * NS * EPT2

BS = 1000
NBLK = N // BS

_mesh = plsc.VectorSubcoreMesh(
    core_axis_name="c", subcore_axis_name="s", num_cores=NC, num_subcores=NS
)


_RCHUNKS = [(0, CH), (CH, CH), (2 * CH, CH), (3 * CH, CH), (4 * CH, RPT - 4 * CH)]


def _zero_spmem(rows, acc, rbase):
    for off, n in _RCHUNKS:
        pltpu.sync_copy(rows.at[pl.ds(0, n)], acc.at[pl.ds(rbase + off, n)])


def _writeout_spmem(rows, acc, rbase, out_slice_fn):
    for off, n in _RCHUNKS:
        pltpu.sync_copy(acc.at[pl.ds(rbase + off, n)], rows.at[pl.ds(0, n)])
        pltpu.sync_copy(rows.at[pl.ds(0, n)], out_slice_fn(rbase + off, n))


def _sc1_body(table, srcs2, dsts, zrows, zdeg, ones1,
              agg_out, deg_out,
              s_idx, d_idx, rows, ones_r, dbuf, acc, dacc, sem):
    c = lax.axis_index("c")
    s = lax.axis_index("s")
    rbase = pl.multiple_of(s * RPT, 8)
    pltpu.sync_copy(zrows, rows)
    _zero_spmem(rows, acc, rbase)
    pltpu.sync_copy(zdeg, dbuf)
    pltpu.sync_copy(dbuf, dacc.at[pl.ds(rbase, RPT)])
    pltpu.sync_copy(ones1, ones_r)
    plsc.subcore_barrier()

    ebase = s * EPT1

    def step(j, carry):
        off = pl.multiple_of(ebase + j * CH, 8)
        pltpu.sync_copy(srcs2.at[pl.ds(c * EPAD1 + off, CH)], s_idx)
        pltpu.sync_copy(dsts.at[pl.ds(off, CH)], d_idx)
        pltpu.async_copy(table.at[s_idx], rows, sem).wait()
        pltpu.sync_copy(rows, acc.at[d_idx], add=True)
        pltpu.sync_copy(ones_r, dacc.at[d_idx], add=True)
        return carry

    lax.fori_loop(0, EPT1 // CH, step, 0)
    plsc.subcore_barrier()

    _writeout_spmem(rows, acc, rbase, lambda r, n: agg_out.at[c, pl.ds(r, n)])

    @pl.when(c == 0)
    def _():
        pltpu.sync_copy(dacc.at[pl.ds(rbase, RPT)], dbuf)
        pltpu.sync_copy(dbuf, deg_out.at[pl.ds(rbase, RPT)])


@functools.cache
def _sc1():
  return pl.kernel(
    _sc1_body,
    out_type=(
        jax.ShapeDtypeStruct((NC, NP, 128), jnp.float32),
        jax.ShapeDtypeStruct((NP,), jnp.float32),
    ),
    mesh=_mesh,
    scratch_types=[
        pltpu.VMEM((CH,), jnp.int32),
        pltpu.VMEM((CH,), jnp.int32),
        pltpu.VMEM((CH, 128), jnp.float32),
        pltpu.VMEM((CH,), jnp.float32),
        pltpu.VMEM((RPT,), jnp.float32),
        pltpu.VMEM_SHARED((NP, 128), jnp.float32),
        pltpu.VMEM_SHARED((NP,), jnp.float32),
        pltpu.SemaphoreType.DMA,
    ],
  )


def _sc2_body(table, srcs, dsts, zrows,
              agg_out,
              s_idx, d_idx, rows, acc, sem):
    c = lax.axis_index("c")
    s = lax.axis_index("s")
    rbase = pl.multiple_of(s * RPT, 8)
    pltpu.sync_copy(zrows, rows)
    _zero_spmem(rows, acc, rbase)
    plsc.subcore_barrier()

    ebase = (c * NS + s) * EPT2

    def step(j, carry):
        off = pl.multiple_of(ebase + j * CH, 8)
        pltpu.sync_copy(srcs.at[pl.ds(off, CH)], s_idx)
        pltpu.sync_copy(dsts.at[pl.ds(off, CH)], d_idx)
        pltpu.async_copy(table.at[s_idx], rows, sem).wait()
        pltpu.sync_copy(rows, acc.at[d_idx], add=True)
        return carry

    lax.fori_loop(0, EPT2 // CH, step, 0)
    plsc.subcore_barrier()

    _writeout_spmem(rows, acc, rbase, lambda r, n: agg_out.at[c, pl.ds(r, n)])


@functools.cache
def _sc2():
  return pl.kernel(
    _sc2_body,
    out_type=jax.ShapeDtypeStruct((NC, NP, 128), jnp.float32),
    mesh=_mesh,
    scratch_types=[
        pltpu.VMEM((CH,), jnp.int32),
        pltpu.VMEM((CH,), jnp.int32),
        pltpu.VMEM((CH, 128), jnp.float32),
        pltpu.VMEM_SHARED((NP, 128), jnp.float32),
        pltpu.SemaphoreType.DMA,
    ],
  )


def _k1_body(x_ref, w_ref, out_ref):
    out_ref[...] = jnp.dot(
        x_ref[...], w_ref[...], preferred_element_type=jnp.float32
    )[None]


def _k1(x, W1):
    return pl.pallas_call(
        _k1_body,
        grid=(2, NBLK),
        in_specs=[
            pl.BlockSpec((BS, NFEAT), lambda c, nb: (nb, 0)),
            pl.BlockSpec((NFEAT, 128), lambda c, nb: (0, c)),
        ],
        out_specs=pl.BlockSpec((1, BS, 128), lambda c, nb: (c, nb, 0)),
        out_shape=jax.ShapeDtypeStruct((2, N, 128), jnp.float32),
    )(x, W1)


def _k2_body(a0_ref, a1_ref, s0_ref, s1_ref, deg_ref, b1_ref, w2_ref, out_ref):
    inv = 1.0 / (deg_ref[...] + 1.0)
    h0 = jnp.maximum((a0_ref[0] + s0_ref[0]) * inv + b1_ref[:, :128], 0.0)
    h1 = jnp.maximum((a1_ref[0] + s1_ref[0]) * inv + b1_ref[:, 128:], 0.0)
    h = jnp.concatenate([h0, h1], axis=1)
    out_ref[...] = jnp.dot(h, w2_ref[...], preferred_element_type=jnp.float32)


def _k2(agg1, sup1, deg, b1, W2):
    return pl.pallas_call(
        _k2_body,
        grid=(NBLK,),
        in_specs=[
            pl.BlockSpec((1, BS, 128), lambda nb: (0, nb, 0)),
            pl.BlockSpec((1, BS, 128), lambda nb: (1, nb, 0)),
            pl.BlockSpec((1, BS, 128), lambda nb: (0, nb, 0)),
            pl.BlockSpec((1, BS, 128), lambda nb: (1, nb, 0)),
            pl.BlockSpec((BS, 1), lambda nb: (nb, 0)),
            pl.BlockSpec((1, NHID), lambda nb: (0, 0)),
            pl.BlockSpec((NHID, 128), lambda nb: (0, 0)),
        ],
        out_specs=pl.BlockSpec((BS, 128), lambda nb: (nb, 0)),
        out_shape=jax.ShapeDtypeStruct((N, 128), jnp.float32),
    )(agg1, agg1, sup1, sup1, deg, b1, W2)


def _k3_body(p0_ref, p1_ref, s_ref, deg_ref, b2_ref, out_ref):
    inv = 1.0 / (deg_ref[...] + 1.0)
    o = (p0_ref[0, :, :64] + p1_ref[0, :, :64] + s_ref[:, :64]) * inv + b2_ref[...]
    m = jnp.max(o, axis=1, keepdims=True)
    e = o - m
    lse = jnp.log(jnp.sum(jnp.exp(e), axis=1, keepdims=True))
    out_ref[...] = e - lse


def _k3(agg2, sup2, deg, b2):
    return pl.pallas_call(
        _k3_body,
        grid=(NBLK,),
        in_specs=[
            pl.BlockSpec((1, BS, 128), lambda nb: (0, nb, 0)),
            pl.BlockSpec((1, BS, 128), lambda nb: (1, nb, 0)),
            pl.BlockSpec((BS, 128), lambda nb: (nb, 0)),
            pl.BlockSpec((BS, 1), lambda nb: (nb, 0)),
            pl.BlockSpec((1, NCLASS), lambda nb: (0, 0)),
        ],
        out_specs=pl.BlockSpec((BS, NCLASS), lambda nb: (nb, 0)),
        out_shape=jax.ShapeDtypeStruct((N, NCLASS), jnp.float32),
    )(agg2, agg2, sup2, deg, b2)


def _pad_edges(arr, n_parts, ept, fill):
    per = E // n_parts
    a = arr.reshape(n_parts, per)
    return jnp.pad(a, ((0, 0), (0, ept - per)), constant_values=fill).reshape(-1)


@jax.jit
def kernel(x, adj, W1, b1, W2, b2):
    src = adj[0].astype(jnp.int32)
    dst = adj[1].astype(jnp.int32)

    src1 = _pad_edges(src, NS, EPT1, 0)
    dst1 = _pad_edges(dst, NS, EPT1, N)
    srcs2 = jnp.concatenate([src1, src1 + N])
    src2 = _pad_edges(src, NC * NS, EPT2, 0)
    dst2 = _pad_edges(dst, NC * NS, EPT2, N)

    zrows1 = jnp.zeros((CH, 128), jnp.float32)
    zrows2 = jnp.zeros((CH, 128), jnp.float32)
    zdeg = jnp.zeros((RPT,), jnp.float32)
    ones1 = jnp.ones((CH,), jnp.float32)

    sup1 = _k1(x, W1)
    table1 = sup1.reshape(2 * N, 128)
    agg1, deg = _sc1()(table1, srcs2, dst1, zrows1, zdeg, ones1)
    deg_col = deg[:N].reshape(N, 1)
    b1r = b1.reshape(1, NHID)

    W2p = jnp.pad(W2, ((0, 0), (0, 128 - NCLASS)))
    sup2 = _k2(agg1, sup1, deg_col, b1r, W2p)
    agg2 = _sc2()(sup2, src2, dst2, zrows2)
    return _k3(agg2, sup2, deg_col, b2.reshape(1, NCLASS))


# speedup vs baseline: 3.9962x; 1.1513x over previous
import functools

import jax
import jax.numpy as jnp
from jax import lax
from jax.experimental import pallas as pl
from jax.experimental.pallas import tpu as pltpu
from jax.experimental.pallas import tpu_sc as plsc

N = 10000
E = 160000
NFEAT = 256
NHID = 256
NCLASS = 64

NC = 2
NS = 16
CH = 128

NP = 10112
RPT = NP // NS

NCH1 = 80
EPT1 = NCH1 * CH
EPAD1 = NS * EPT1

NCH2 = 40
EPT2 = NCH2 * CH
EPAD2 = NC * NS * EPT2

BS = 1000
NBLK = N // BS

_mesh = plsc.VectorSubcoreMesh(
    core_axis_name="c", subcore_axis_name="s", num_cores=NC, num_subcores=NS
)


_RCHUNKS = [(0, CH), (CH, CH), (2 * CH, CH), (3 * CH, CH), (4 * CH, RPT - 4 * CH)]


def _zero_spmem(rows, acc, rbase):
    for off, n in _RCHUNKS:
        pltpu.sync_copy(rows.at[pl.ds(0, n)], acc.at[pl.ds(rbase + off, n)])


def _writeout_spmem(rows, acc, rbase, out_slice_fn):
    for off, n in _RCHUNKS:
        pltpu.sync_copy(acc.at[pl.ds(rbase + off, n)], rows.at[pl.ds(0, n)])
        pltpu.sync_copy(rows.at[pl.ds(0, n)], out_slice_fn(rbase + off, n))


def _sc1_body(table, srcs3, dsts3, zrows, zdeg, ones1,
              agg_out, deg_out,
              sidx0, sidx1, dbuf, rows_a, rows_b, ones_r, degb, acc, dacc,
              gsa, gsb, ssa, ssb, dsa, dsb, isem):
    c = lax.axis_index("c")
    s = lax.axis_index("s")
    rbase = pl.multiple_of(s * RPT, 8)
    w = c * NS + s
    pltpu.sync_copy(dsts3.at[s], dbuf)
    pltpu.sync_copy(srcs3.at[w, 0], sidx0)
    pltpu.sync_copy(srcs3.at[w, 1], sidx1)
    pltpu.sync_copy(zrows, rows_a)
    _zero_spmem(rows_a, acc, rbase)
    pltpu.sync_copy(zdeg, degb)
    pltpu.sync_copy(degb, dacc.at[pl.ds(rbase, RPT)])
    pltpu.sync_copy(ones1, ones_r)
    plsc.subcore_barrier()

    pltpu.async_copy(table.at[sidx0], rows_a, gsa)

    def pair(i, carry):
        j0 = i * 2
        j1 = j0 + 1
        pltpu.make_async_copy(table.at[sidx0], rows_a, gsa).wait()
        pltpu.async_copy(rows_a, acc.at[dbuf.at[j0]], ssa, add=True)
        pltpu.async_copy(ones_r, dacc.at[dbuf.at[j0]], dsa, add=True)
        jn0 = jnp.where(j0 + 2 < NCH1, j0 + 2, 0)
        pltpu.async_copy(srcs3.at[w, jn0], sidx0, isem)

        @pl.when(i > 0)
        def _():
            pltpu.make_async_copy(rows_b, acc.at[dbuf.at[0]], ssb).wait()
            pltpu.make_async_copy(ones_r, dacc.at[dbuf.at[0]], dsb).wait()

        pltpu.async_copy(table.at[sidx1], rows_b, gsb)
        pltpu.make_async_copy(table.at[sidx1], rows_b, gsb).wait()
        pltpu.async_copy(rows_b, acc.at[dbuf.at[j1]], ssb, add=True)
        pltpu.async_copy(ones_r, dacc.at[dbuf.at[j1]], dsb, add=True)
        jn1 = jnp.where(j1 + 2 < NCH1, j1 + 2, 0)
        pltpu.async_copy(srcs3.at[w, jn1], sidx1, isem)

        pltpu.make_async_copy(rows_a, acc.at[dbuf.at[0]], ssa).wait()
        pltpu.make_async_copy(ones_r, dacc.at[dbuf.at[0]], dsa).wait()
        pltpu.make_async_copy(srcs3.at[w, 0], sidx0, isem).wait()
        pltpu.async_copy(table.at[sidx0], rows_a, gsa)
        pltpu.make_async_copy(srcs3.at[w, 0], sidx1, isem).wait()
        return carry

    lax.fori_loop(0, NCH1 // 2, pair, 0)
    pltpu.make_async_copy(table.at[sidx0], rows_a, gsa).wait()
    pltpu.make_async_copy(rows_b, acc.at[dbuf.at[0]], ssb).wait()
    pltpu.make_async_copy(ones_r, dacc.at[dbuf.at[0]], dsb).wait()
    plsc.subcore_barrier()

    _writeout_spmem(rows_a, acc, rbase, lambda r, n: agg_out.at[c, pl.ds(r, n)])

    @pl.when(c == 0)
    def _():
        pltpu.sync_copy(dacc.at[pl.ds(rbase, RPT)], degb)
        pltpu.sync_copy(degb, deg_out.at[pl.ds(rbase, RPT)])


@functools.cache
def _sc1():
  return pl.kernel(
    _sc1_body,
    out_type=(
        jax.ShapeDtypeStruct((NC, NP, 128), jnp.float32),
        jax.ShapeDtypeStruct((NP,), jnp.float32),
    ),
    mesh=_mesh,
    scratch_types=[
        pltpu.VMEM((CH,), jnp.int32),
        pltpu.VMEM((CH,), jnp.int32),
        pltpu.VMEM((NCH1, CH), jnp.int32),
        pltpu.VMEM((CH, 128), jnp.float32),
        pltpu.VMEM((CH, 128), jnp.float32),
        pltpu.VMEM((CH,), jnp.float32),
        pltpu.VMEM((RPT,), jnp.float32),
        pltpu.VMEM_SHARED((NP, 128), jnp.float32),
        pltpu.VMEM_SHARED((NP,), jnp.float32),
        pltpu.SemaphoreType.DMA,
        pltpu.SemaphoreType.DMA,
        pltpu.SemaphoreType.DMA,
        pltpu.SemaphoreType.DMA,
        pltpu.SemaphoreType.DMA,
        pltpu.SemaphoreType.DMA,
        pltpu.SemaphoreType.DMA,
    ],
  )


def _sc2_body(table, srcs3, dsts3, zrows,
              agg_out,
              sidx0, sidx1, dbuf, rows_a, rows_b, acc,
              gsa, gsb, ssa, ssb, isem):
    c = lax.axis_index("c")
    s = lax.axis_index("s")
    rbase = pl.multiple_of(s * RPT, 8)
    w = c * NS + s
    pltpu.sync_copy(dsts3.at[w], dbuf)
    pltpu.sync_copy(srcs3.at[w, 0], sidx0)
    pltpu.sync_copy(srcs3.at[w, 1], sidx1)
    pltpu.sync_copy(zrows, rows_a)
    _zero_spmem(rows_a, acc, rbase)
    plsc.subcore_barrier()

    pltpu.async_copy(table.at[sidx0], rows_a, gsa)

    def pair(i, carry):
        j0 = i * 2
        j1 = j0 + 1
        pltpu.make_async_copy(table.at[sidx0], rows_a, gsa).wait()
        pltpu.async_copy(rows_a, acc.at[dbuf.at[j0]], ssa, add=True)
        jn0 = jnp.where(j0 + 2 < NCH2, j0 + 2, 0)
        pltpu.async_copy(srcs3.at[w, jn0], sidx0, isem)

        @pl.when(i > 0)
        def _():
            pltpu.make_async_copy(rows_b, acc.at[dbuf.at[0]], ssb).wait()

        pltpu.async_copy(table.at[sidx1], rows_b, gsb)
        pltpu.make_async_copy(table.at[sidx1], rows_b, gsb).wait()
        pltpu.async_copy(rows_b, acc.at[dbuf.at[j1]], ssb, add=True)
        jn1 = jnp.where(j1 + 2 < NCH2, j1 + 2, 0)
        pltpu.async_copy(srcs3.at[w, jn1], sidx1, isem)

        pltpu.make_async_copy(rows_a, acc.at[dbuf.at[0]], ssa).wait()
        pltpu.make_async_copy(srcs3.at[w, 0], sidx0, isem).wait()
        pltpu.async_copy(table.at[sidx0], rows_a, gsa)
        pltpu.make_async_copy(srcs3.at[w, 0], sidx1, isem).wait()
        return carry

    lax.fori_loop(0, NCH2 // 2, pair, 0)
    pltpu.make_async_copy(table.at[sidx0], rows_a, gsa).wait()
    pltpu.make_async_copy(rows_b, acc.at[dbuf.at[0]], ssb).wait()
    plsc.subcore_barrier()

    _writeout_spmem(rows_a, acc, rbase, lambda r, n: agg_out.at[c, pl.ds(r, n)])


@functools.cache
def _sc2():
  return pl.kernel(
    _sc2_body,
    out_type=jax.ShapeDtypeStruct((NC, NP, 128), jnp.float32),
    mesh=_mesh,
    scratch_types=[
        pltpu.VMEM((CH,), jnp.int32),
        pltpu.VMEM((CH,), jnp.int32),
        pltpu.VMEM((NCH2, CH), jnp.int32),
        pltpu.VMEM((CH, 128), jnp.float32),
        pltpu.VMEM((CH, 128), jnp.float32),
        pltpu.VMEM_SHARED((NP, 128), jnp.float32),
        pltpu.SemaphoreType.DMA,
        pltpu.SemaphoreType.DMA,
        pltpu.SemaphoreType.DMA,
        pltpu.SemaphoreType.DMA,
        pltpu.SemaphoreType.DMA,
    ],
  )


def _k1_body(x_ref, w_ref, out_ref):
    out_ref[...] = jnp.dot(
        x_ref[...], w_ref[...], preferred_element_type=jnp.float32
    )[None]


def _k1(x, W1):
    return pl.pallas_call(
        _k1_body,
        grid=(2, NBLK),
        in_specs=[
            pl.BlockSpec((BS, NFEAT), lambda c, nb: (nb, 0)),
            pl.BlockSpec((NFEAT, 128), lambda c, nb: (0, c)),
        ],
        out_specs=pl.BlockSpec((1, BS, 128), lambda c, nb: (c, nb, 0)),
        out_shape=jax.ShapeDtypeStruct((2, N, 128), jnp.float32),
    )(x, W1)


def _k2_body(a0_ref, a1_ref, s0_ref, s1_ref, deg_ref, b1_ref, w2_ref, out_ref):
    inv = 1.0 / (deg_ref[...] + 1.0)
    h0 = jnp.maximum((a0_ref[0] + s0_ref[0]) * inv + b1_ref[:, :128], 0.0)
    h1 = jnp.maximum((a1_ref[0] + s1_ref[0]) * inv + b1_ref[:, 128:], 0.0)
    h = jnp.concatenate([h0, h1], axis=1)
    out_ref[...] = jnp.dot(h, w2_ref[...], preferred_element_type=jnp.float32)


def _k2(agg1, sup1, deg, b1, W2):
    return pl.pallas_call(
        _k2_body,
        grid=(NBLK,),
        in_specs=[
            pl.BlockSpec((1, BS, 128), lambda nb: (0, nb, 0)),
            pl.BlockSpec((1, BS, 128), lambda nb: (1, nb, 0)),
            pl.BlockSpec((1, BS, 128), lambda nb: (0, nb, 0)),
            pl.BlockSpec((1, BS, 128), lambda nb: (1, nb, 0)),
            pl.BlockSpec((BS, 1), lambda nb: (nb, 0)),
            pl.BlockSpec((1, NHID), lambda nb: (0, 0)),
            pl.BlockSpec((NHID, 128), lambda nb: (0, 0)),
        ],
        out_specs=pl.BlockSpec((BS, 128), lambda nb: (nb, 0)),
        out_shape=jax.ShapeDtypeStruct((N, 128), jnp.float32),
    )(agg1, agg1, sup1, sup1, deg, b1, W2)


def _k3_body(p0_ref, p1_ref, s_ref, deg_ref, b2_ref, out_ref):
    inv = 1.0 / (deg_ref[...] + 1.0)
    o = (p0_ref[0, :, :64] + p1_ref[0, :, :64] + s_ref[:, :64]) * inv + b2_ref[...]
    m = jnp.max(o, axis=1, keepdims=True)
    e = o - m
    lse = jnp.log(jnp.sum(jnp.exp(e), axis=1, keepdims=True))
    out_ref[...] = e - lse


def _k3(agg2, sup2, deg, b2):
    return pl.pallas_call(
        _k3_body,
        grid=(NBLK,),
        in_specs=[
            pl.BlockSpec((1, BS, 128), lambda nb: (0, nb, 0)),
            pl.BlockSpec((1, BS, 128), lambda nb: (1, nb, 0)),
            pl.BlockSpec((BS, 128), lambda nb: (nb, 0)),
            pl.BlockSpec((BS, 1), lambda nb: (nb, 0)),
            pl.BlockSpec((1, NCLASS), lambda nb: (0, 0)),
        ],
        out_specs=pl.BlockSpec((BS, NCLASS), lambda nb: (nb, 0)),
        out_shape=jax.ShapeDtypeStruct((N, NCLASS), jnp.float32),
    )(agg2, agg2, sup2, deg, b2)


def _pad_edges(arr, n_parts, ept, fill):
    per = E // n_parts
    a = arr.reshape(n_parts, per)
    return jnp.pad(a, ((0, 0), (0, ept - per)), constant_values=fill).reshape(-1)


@jax.jit
def kernel(x, adj, W1, b1, W2, b2):
    src = adj[0].astype(jnp.int32)
    dst = adj[1].astype(jnp.int32)

    src1 = _pad_edges(src, NS, EPT1, 0).reshape(NS, NCH1, CH)
    dst1 = _pad_edges(dst, NS, EPT1, N).reshape(NS, NCH1, CH)
    srcs2 = jnp.concatenate([src1, src1 + N])
    src2 = _pad_edges(src, NC * NS, EPT2, 0).reshape(NC * NS, NCH2, CH)
    dst2 = _pad_edges(dst, NC * NS, EPT2, N).reshape(NC * NS, NCH2, CH)

    zrows1 = jnp.zeros((CH, 128), jnp.float32)
    zrows2 = jnp.zeros((CH, 128), jnp.float32)
    zdeg = jnp.zeros((RPT,), jnp.float32)
    ones1 = jnp.ones((CH,), jnp.float32)

    sup1 = _k1(x, W1)
    table1 = sup1.reshape(2 * N, 128)
    agg1, deg = _sc1()(table1, srcs2, dst1, zrows1, zdeg, ones1)
    deg_col = deg[:N].reshape(N, 1)
    b1r = b1.reshape(1, NHID)

    W2p = jnp.pad(W2, ((0, 0), (0, 128 - NCLASS)))
    sup2 = _k2(agg1, sup1, deg_col, b1r, W2p)
    agg2 = _sc2()(sup2, src2, dst2, zrows2)
    return _k3(agg2, sup2, deg_col, b2.reshape(1, NCLASS))
